# Initial kernel scaffold; baseline (speedup 1.0000x reference)
import jax
import jax.numpy as jnp
from jax.experimental import pallas as pl


def kernel(t, c, n, t_weight, c_weight):
    raise NotImplementedError("write your pallas kernel here")


# speedup vs baseline: 4.7604x; 4.7604x over previous
import functools

import jax
import jax.numpy as jnp
from jax import lax
from jax.experimental import pallas as pl
from jax.experimental.pallas import tpu as pltpu
from jax.experimental.pallas import tpu_sc as plsc

_V = 1000000
_D = 64
_B = 16384
_K = 20
_J = _K + 1

_NC = 2
_NS = 16
_NW = _NC * _NS
_PER_W = _B // _NW
_E = 64
_NCHUNK = _PER_W // _E
_ROWS = _E * _J
_SUB = 112
_NSUB = _ROWS // _SUB


def _sc_scores(t, cn2, t_weight, c_weight):
    mesh = plsc.VectorSubcoreMesh(core_axis_name="c", subcore_axis_name="s")

    @functools.partial(
        pl.kernel,
        mesh=mesh,
        compiler_params=pltpu.CompilerParams(
            needs_layout_passes=False, use_tc_tiling_on_sc=False),
        out_type=jax.ShapeDtypeStruct((_B * _J,), jnp.float32),
        scratch_types=[
            pltpu.VMEM((_E,), jnp.int32),
            pltpu.VMEM((_ROWS,), jnp.int32),
            pltpu.VMEM((_E, _D), jnp.float32),
            pltpu.VMEM((_ROWS, _D), jnp.float32),
            pltpu.VMEM((_PER_W * _J,), jnp.float32),
            pltpu.SemaphoreType.DMA,
        ],
    )
    def k(t_hbm, cn_hbm, tw_hbm, cw_hbm, out_hbm,
          idx_t, idx_cn, vt_rows, cn_rows, scores_v, sem):
        wid = lax.axis_index("s") * _NC + lax.axis_index("c")
        base = wid * _PER_W
        lane = lax.iota(jnp.int32, 16)
        m15 = lane == 15

        def chunk_body(ci, carry):
            e0 = base + ci * _E
            pltpu.sync_copy(t_hbm.at[pl.ds(e0, _E)], idx_t)
            pltpu.sync_copy(cn_hbm.at[pl.ds(e0 * _J, _ROWS)], idx_cn)
            cp_t = pltpu.async_copy(tw_hbm.at[idx_t], vt_rows, sem)
            cps = [
                pltpu.async_copy(cw_hbm.at[idx_cn.at[pl.ds(j * _SUB, _SUB)]],
                                 cn_rows.at[pl.ds(j * _SUB, _SUB)], sem)
                for j in range(_NSUB)
            ]
            cp_t.wait()
            for cp in cps:
                cp.wait()

            def e_body(e, c2):
                ge = ci * _E + e
                base_idx = jnp.full((16,), ge * _J, jnp.int32)
                vt = [vt_rows[e, pl.ds(s * 16, 16)] for s in range(4)]
                for j in range(_J):
                    r = e * _J + j
                    acc = vt[0] * cn_rows[r, pl.ds(0, 16)]
                    for s in range(1, 4):
                        acc = acc + vt[s] * cn_rows[r, pl.ds(s * 16, 16)]
                    if j == 0:
                        acc = -acc
                    sval = jnp.sum(acc)
                    csum = jnp.full((16,), sval, jnp.float32)
                    plsc.store_scatter(scores_v, [base_idx + j], csum,
                                       mask=m15)
                return c2

            lax.fori_loop(0, _E, e_body, 0)
            return carry

        lax.fori_loop(0, _NCHUNK, chunk_body, 0)
        pltpu.sync_copy(scores_v, out_hbm.at[pl.ds(base * _J, _PER_W * _J)])

    return k(t, cn2, t_weight, c_weight)


def _loss_tc(scores):
    flat = scores.reshape(_B * _J // 128, 128)

    def body(s_ref, o_ref):
        x = s_ref[...]
        term = -jnp.log(jax.nn.sigmoid(-x) + 1e-10)
        o_ref[0, 0] = jnp.sum(term) * (1.0 / _B)

    out = pl.pallas_call(
        body,
        out_shape=jax.ShapeDtypeStruct((1, 1), jnp.float32),
        out_specs=pl.BlockSpec(memory_space=pltpu.SMEM),
    )(flat)
    return out[0, 0]


def kernel(t, c, n, t_weight, c_weight):
    t = t.astype(jnp.int32)
    cn = jnp.concatenate(
        [c.astype(jnp.int32)[:, None], n.astype(jnp.int32)], axis=1)
    cn2 = cn.reshape(_B * _J)
    scores = _sc_scores(t, cn2, t_weight, c_weight)
    return _loss_tc(scores)


# speedup vs baseline: 7.6773x; 1.6127x over previous
import functools

import jax
import jax.numpy as jnp
from jax import lax
from jax.experimental import pallas as pl
from jax.experimental.pallas import tpu as pltpu
from jax.experimental.pallas import tpu_sc as plsc

_V = 1000000
_D = 64
_B = 16384
_K = 20
_J = _K + 1

_NC = 2
_NS = 16
_NW = _NC * _NS
_PER_W = _B // _NW
_E = 32
_NCHUNK = _PER_W // _E
_ROWS = _E * _J
_SUB = 112
_NSUB = _ROWS // _SUB

_BLKI = 4096


def _expand_tables(t_wt, c_wt):
    tT = t_wt.T
    cT = c_wt.T

    def body(t_in, c_in, tp, cp):
        tp[:, 0:_D] = t_in[...].T
        cp[:, 0:_D] = c_in[...].T

    in_spec = pl.BlockSpec((_D, _BLKI), lambda i: (0, i))
    out_spec = pl.BlockSpec((_BLKI, 2 * _D), lambda i: (i, 0))
    return pl.pallas_call(
        body,
        grid=((_V + _BLKI - 1) // _BLKI,),
        in_specs=[in_spec, in_spec],
        out_specs=[out_spec, out_spec],
        out_shape=[jax.ShapeDtypeStruct((_V, 2 * _D), jnp.float32)] * 2,
    )(tT, cT)


def _sc_scores(t, cn, tp, cp):
    mesh = plsc.VectorSubcoreMesh(core_axis_name="c", subcore_axis_name="s")

    @functools.partial(
        pl.kernel,
        mesh=mesh,
        compiler_params=pltpu.CompilerParams(needs_layout_passes=False),
        out_type=jax.ShapeDtypeStruct((_B * _J,), jnp.float32),
        scratch_types=[
            pltpu.VMEM((_E,), jnp.int32),
            pltpu.VMEM((_ROWS,), jnp.int32),
            pltpu.VMEM((_E, 2 * _D), jnp.float32),
            pltpu.VMEM((_ROWS, 2 * _D), jnp.float32),
            pltpu.VMEM((_PER_W * _J,), jnp.float32),
            pltpu.SemaphoreType.DMA,
        ],
    )
    def k(t_hbm, cn_hbm, tp_hbm, cp_hbm, out_hbm,
          idx_t, idx_cn, vt_rows, cn_rows, scores_v, sem):
        wid = lax.axis_index("s") * _NC + lax.axis_index("c")
        base = wid * _PER_W
        lane = lax.iota(jnp.int32, 16)
        m15 = lane == 15

        def chunk_body(ci, carry):
            e0 = base + ci * _E
            pltpu.sync_copy(t_hbm.at[pl.ds(e0, _E)], idx_t)
            pltpu.sync_copy(cn_hbm.at[pl.ds(e0 * _J, _ROWS)], idx_cn)
            cp_t = pltpu.async_copy(tp_hbm.at[idx_t], vt_rows, sem)
            cps = [
                pltpu.async_copy(cp_hbm.at[idx_cn.at[pl.ds(j * _SUB, _SUB)]],
                                 cn_rows.at[pl.ds(j * _SUB, _SUB)], sem)
                for j in range(_NSUB)
            ]
            cp_t.wait()
            for cp in cps:
                cp.wait()

            def e_body(e, c2):
                ge = ci * _E + e
                base_idx = jnp.full((16,), ge * _J, jnp.int32)
                vt = [vt_rows[e, pl.ds(s * 16, 16)] for s in range(4)]
                for j in range(_J):
                    r = e * _J + j
                    acc = vt[0] * cn_rows[r, pl.ds(0, 16)]
                    for s in range(1, 4):
                        acc = acc + vt[s] * cn_rows[r, pl.ds(s * 16, 16)]
                    if j == 0:
                        acc = -acc
                    sval = jnp.sum(acc)
                    csum = jnp.full((16,), sval, jnp.float32)
                    plsc.store_scatter(scores_v, [base_idx + j], csum,
                                       mask=m15)
                return c2

            lax.fori_loop(0, _E, e_body, 0)
            return carry

        lax.fori_loop(0, _NCHUNK, chunk_body, 0)
        pltpu.sync_copy(scores_v, out_hbm.at[pl.ds(base * _J, _PER_W * _J)])

    return k(t, cn, tp, cp)


def _loss_tc(scores):
    flat = scores.reshape(_B * _J // 128, 128)

    def body(s_ref, o_ref):
        x = s_ref[...]
        term = -jnp.log(jax.nn.sigmoid(-x) + 1e-10)
        o_ref[0, 0] = jnp.sum(term) * (1.0 / _B)

    out = pl.pallas_call(
        body,
        out_shape=jax.ShapeDtypeStruct((1, 1), jnp.float32),
        out_specs=pl.BlockSpec(memory_space=pltpu.SMEM),
    )(flat)
    return out[0, 0]


def kernel(t, c, n, t_weight, c_weight):
    t = t.astype(jnp.int32)
    cn = jnp.concatenate(
        [c.astype(jnp.int32)[:, None], n.astype(jnp.int32)], axis=1
    ).reshape(_B * _J)
    tp, cp = _expand_tables(t_weight, c_weight)
    scores = _sc_scores(t, cn, tp, cp)
    return _loss_tc(scores)


# speedup vs baseline: 7.9682x; 1.0379x over previous
import functools

import jax
import jax.numpy as jnp
from jax import lax
from jax.experimental import pallas as pl
from jax.experimental.pallas import tpu as pltpu
from jax.experimental.pallas import tpu_sc as plsc

_V = 1000000
_D = 64
_B = 16384
_K = 20
_J = _K + 1

_NC = 2
_NS = 16
_NW = _NC * _NS
_PER_W = _B // _NW
_E = 32
_NCHUNK = _PER_W // _E
_ROWS = _E * _J
_SUB = 112
_NSUB = _ROWS // _SUB

_BLKI = 4096


def _expand_tables(t_wt, c_wt):
    tT = t_wt.T
    cT = c_wt.T

    def body(t_in, c_in, tp, cp):
        tp[:, 0:_D] = t_in[...].T
        cp[:, 0:_D] = c_in[...].T

    in_spec = pl.BlockSpec((_D, _BLKI), lambda i: (0, i))
    out_spec = pl.BlockSpec((_BLKI, 2 * _D), lambda i: (i, 0))
    return pl.pallas_call(
        body,
        grid=((_V + _BLKI - 1) // _BLKI,),
        in_specs=[in_spec, in_spec],
        out_specs=[out_spec, out_spec],
        out_shape=[jax.ShapeDtypeStruct((_V, 2 * _D), jnp.float32)] * 2,
    )(tT, cT)


def _sc_scores(t, cn, tp, cp):
    mesh = plsc.VectorSubcoreMesh(core_axis_name="c", subcore_axis_name="s")

    @functools.partial(
        pl.kernel,
        mesh=mesh,
        compiler_params=pltpu.CompilerParams(
            needs_layout_passes=False, use_tc_tiling_on_sc=False),
        out_type=jax.ShapeDtypeStruct((_B * _J,), jnp.float32),
        scratch_types=[
            pltpu.VMEM((_E,), jnp.int32),
            pltpu.VMEM((_ROWS,), jnp.int32),
            pltpu.VMEM((_E, _D), jnp.float32),
            pltpu.VMEM((_ROWS, _D), jnp.float32),
            pltpu.VMEM((_PER_W * _J,), jnp.float32),
            pltpu.SemaphoreType.DMA,
        ],
    )
    def k(t_hbm, cn_hbm, tp_hbm, cp_hbm, out_hbm,
          idx_t, idx_cn, vt_rows, cn_rows, scores_v, sem):
        wid = lax.axis_index("s") * _NC + lax.axis_index("c")
        base = wid * _PER_W
        lane = lax.iota(jnp.int32, 16)
        m15 = lane == 15

        def chunk_body(ci, carry):
            e0 = base + ci * _E
            pltpu.sync_copy(t_hbm.at[pl.ds(e0, _E)], idx_t)
            pltpu.sync_copy(cn_hbm.at[pl.ds(e0 * _J, _ROWS)], idx_cn)
            cp_t = pltpu.async_copy(tp_hbm.at[idx_t], vt_rows, sem)
            cps = [
                pltpu.async_copy(cp_hbm.at[idx_cn.at[pl.ds(j * _SUB, _SUB)]],
                                 cn_rows.at[pl.ds(j * _SUB, _SUB)], sem)
                for j in range(_NSUB)
            ]
            cp_t.wait()
            for cp in cps:
                cp.wait()

            def e_body(e, c2):
                ge = ci * _E + e
                base_idx = jnp.full((16,), ge * _J, jnp.int32)
                vt = [vt_rows[e, pl.ds(s * 16, 16)] for s in range(4)]
                for j in range(_J):
                    r = e * _J + j
                    acc = vt[0] * cn_rows[r, pl.ds(0, 16)]
                    for s in range(1, 4):
                        acc = acc + vt[s] * cn_rows[r, pl.ds(s * 16, 16)]
                    if j == 0:
                        acc = -acc
                    sval = jnp.sum(acc)
                    csum = jnp.full((16,), sval, jnp.float32)
                    plsc.store_scatter(scores_v, [base_idx + j], csum,
                                       mask=m15)
                return c2

            lax.fori_loop(0, _E, e_body, 0)
            return carry

        lax.fori_loop(0, _NCHUNK, chunk_body, 0)
        pltpu.sync_copy(scores_v, out_hbm.at[pl.ds(base * _J, _PER_W * _J)])

    return k(t, cn, tp, cp)


def _loss_tc(scores):
    flat = scores.reshape(_B * _J // 128, 128)

    def body(s_ref, o_ref):
        x = s_ref[...]
        term = -jnp.log(jax.nn.sigmoid(-x) + 1e-10)
        o_ref[0, 0] = jnp.sum(term) * (1.0 / _B)

    out = pl.pallas_call(
        body,
        out_shape=jax.ShapeDtypeStruct((1, 1), jnp.float32),
        out_specs=pl.BlockSpec(memory_space=pltpu.SMEM),
    )(flat)
    return out[0, 0]


def kernel(t, c, n, t_weight, c_weight):
    t = t.astype(jnp.int32)
    cn = jnp.concatenate(
        [c.astype(jnp.int32)[:, None], n.astype(jnp.int32)], axis=1
    ).reshape(_B * _J)
    tp, cp = _expand_tables(t_weight, c_weight)
    tp2 = tp.reshape(2 * _V, _D)
    cp2 = cp.reshape(2 * _V, _D)
    scores = _sc_scores(t * 2, cn * 2, tp2, cp2)
    return _loss_tc(scores)


# speedup vs baseline: 9.3310x; 1.1710x over previous
import functools

import jax
import jax.numpy as jnp
from jax import lax
from jax.experimental import pallas as pl
from jax.experimental.pallas import tpu as pltpu
from jax.experimental.pallas import tpu_sc as plsc

_V = 1000000
_D = 64
_B = 16384
_K = 20
_J = _K + 1

_NC = 2
_NS = 16
_NW = _NC * _NS
_PER_W = _B // _NW
_E = 32
_NCHUNK = _PER_W // _E
_ROWS = _E * _J
_SUB = 112
_NSUB = _ROWS // _SUB

_BLKO = 7040
_NGRID = 72
_HOUT = _BLKO * _NGRID
_S = 499840
_CLAMP = _V - _BLKO


def _expand_tables(t_wt, c_wt):
    tT = t_wt.T
    cT = c_wt.T

    def body(t_lo, t_hi, c_lo, c_hi, tp, cp):
        tp[:, 0:_D] = t_lo[...].T
        tp[:, _D:2 * _D] = t_hi[...].T
        cp[:, 0:_D] = c_lo[...].T
        cp[:, _D:2 * _D] = c_hi[...].T

    lo_spec = pl.BlockSpec((_D, _BLKO), lambda i: (0, i))
    hi_spec = pl.BlockSpec((_D, _BLKO), lambda i: (0, i + _S // _BLKO))
    out_spec = pl.BlockSpec((_BLKO, 2 * _D), lambda i: (i, 0))
    return pl.pallas_call(
        body,
        grid=(_NGRID,),
        in_specs=[lo_spec, hi_spec, lo_spec, hi_spec],
        out_specs=[out_spec, out_spec],
        out_shape=[jax.ShapeDtypeStruct((_HOUT, 2 * _D), jnp.float32)] * 2,
    )(tT, tT, cT, cT)


def _view_rows(idx):
    lo = 2 * idx
    hi = 2 * (idx - _S) + 1
    tail = 2 * (idx - _CLAMP + _S) + 1
    return jnp.where(idx < _S, lo, jnp.where(idx < _S + 71 * _BLKO, hi, tail))


def _sc_scores(t, cn, tp, cp):
    mesh = plsc.VectorSubcoreMesh(core_axis_name="c", subcore_axis_name="s")

    @functools.partial(
        pl.kernel,
        mesh=mesh,
        compiler_params=pltpu.CompilerParams(
            needs_layout_passes=False, use_tc_tiling_on_sc=False),
        out_type=jax.ShapeDtypeStruct((_B * _J,), jnp.float32),
        scratch_types=[
            pltpu.VMEM((_E,), jnp.int32),
            pltpu.VMEM((_ROWS,), jnp.int32),
            pltpu.VMEM((_E, _D), jnp.float32),
            pltpu.VMEM((_ROWS, _D), jnp.float32),
            pltpu.VMEM((_PER_W * _J,), jnp.float32),
            pltpu.SemaphoreType.DMA,
        ],
    )
    def k(t_hbm, cn_hbm, tp_hbm, cp_hbm, out_hbm,
          idx_t, idx_cn, vt_rows, cn_rows, scores_v, sem):
        wid = lax.axis_index("s") * _NC + lax.axis_index("c")
        base = wid * _PER_W
        lane = lax.iota(jnp.int32, 16)
        m15 = lane == 15

        def chunk_body(ci, carry):
            e0 = base + ci * _E
            pltpu.sync_copy(t_hbm.at[pl.ds(e0, _E)], idx_t)
            pltpu.sync_copy(cn_hbm.at[pl.ds(e0 * _J, _ROWS)], idx_cn)
            cp_t = pltpu.async_copy(tp_hbm.at[idx_t], vt_rows, sem)
            cps = [
                pltpu.async_copy(cp_hbm.at[idx_cn.at[pl.ds(j * _SUB, _SUB)]],
                                 cn_rows.at[pl.ds(j * _SUB, _SUB)], sem)
                for j in range(_NSUB)
            ]
            cp_t.wait()
            for cp in cps:
                cp.wait()

            def e_body(e, c2):
                ge = ci * _E + e
                base_idx = jnp.full((16,), ge * _J, jnp.int32)
                vt = [vt_rows[e, pl.ds(s * 16, 16)] for s in range(4)]
                for j in range(_J):
                    r = e * _J + j
                    acc = vt[0] * cn_rows[r, pl.ds(0, 16)]
                    for s in range(1, 4):
                        acc = acc + vt[s] * cn_rows[r, pl.ds(s * 16, 16)]
                    if j == 0:
                        acc = -acc
                    sval = jnp.sum(acc)
                    csum = jnp.full((16,), sval, jnp.float32)
                    plsc.store_scatter(scores_v, [base_idx + j], csum,
                                       mask=m15)
                return c2

            lax.fori_loop(0, _E, e_body, 0)
            return carry

        lax.fori_loop(0, _NCHUNK, chunk_body, 0)
        pltpu.sync_copy(scores_v, out_hbm.at[pl.ds(base * _J, _PER_W * _J)])

    return k(t, cn, tp, cp)


def _loss_tc(scores):
    flat = scores.reshape(_B * _J // 128, 128)

    def body(s_ref, o_ref):
        x = s_ref[...]
        term = -jnp.log(jax.nn.sigmoid(-x) + 1e-10)
        o_ref[0, 0] = jnp.sum(term) * (1.0 / _B)

    out = pl.pallas_call(
        body,
        out_shape=jax.ShapeDtypeStruct((1, 1), jnp.float32),
        out_specs=pl.BlockSpec(memory_space=pltpu.SMEM),
    )(flat)
    return out[0, 0]


def kernel(t, c, n, t_weight, c_weight):
    t = t.astype(jnp.int32)
    cn = jnp.concatenate(
        [c.astype(jnp.int32)[:, None], n.astype(jnp.int32)], axis=1
    ).reshape(_B * _J)
    tp, cp = _expand_tables(t_weight, c_weight)
    tp2 = tp.reshape(2 * _HOUT, _D)
    cp2 = cp.reshape(2 * _HOUT, _D)
    scores = _sc_scores(_view_rows(t), _view_rows(cn), tp2, cp2)
    return _loss_tc(scores)


# speedup vs baseline: 11.6154x; 1.2448x over previous
import functools

import jax
import jax.numpy as jnp
from jax import lax
from jax.experimental import pallas as pl
from jax.experimental.pallas import tpu as pltpu
from jax.experimental.pallas import tpu_sc as plsc

_V = 1000000
_D = 64
_B = 16384
_K = 20
_J = _K + 1
_G = _J + 1

_NC = 2
_NS = 16
_NW = _NC * _NS
_PER_W = _B // _NW
_E = 32
_NCHUNK = _PER_W // _E
_ROWS = _E * _G
_SUB = 88
_NSUB = _ROWS // _SUB

_BLKI = 8192


def _pack_tables(t_wt, c_wt):
    tT = t_wt.T
    cT = c_wt.T

    def body(t_in, c_in, o):
        x = jnp.concatenate([t_in[...], c_in[...]], axis=0)
        o[...] = x.T

    in_spec = pl.BlockSpec((_D, _BLKI), lambda i: (0, i))
    out_spec = pl.BlockSpec((_BLKI, 2 * _D), lambda i: (i, 0))
    return pl.pallas_call(
        body,
        grid=((_V + _BLKI - 1) // _BLKI,),
        in_specs=[in_spec, in_spec],
        out_specs=out_spec,
        out_shape=jax.ShapeDtypeStruct((_V, 2 * _D), jnp.float32),
    )(tT, cT)


def _sc_scores(gidx, view):
    mesh = plsc.VectorSubcoreMesh(core_axis_name="c", subcore_axis_name="s")

    @functools.partial(
        pl.kernel,
        mesh=mesh,
        compiler_params=pltpu.CompilerParams(
            needs_layout_passes=False, use_tc_tiling_on_sc=False),
        out_type=jax.ShapeDtypeStruct((_B * _J,), jnp.float32),
        scratch_types=[
            pltpu.VMEM((_ROWS,), jnp.int32),
            pltpu.VMEM((_ROWS,), jnp.int32),
            pltpu.VMEM((_ROWS, _D), jnp.float32),
            pltpu.VMEM((_ROWS, _D), jnp.float32),
            pltpu.VMEM((_PER_W * _J,), jnp.float32),
            pltpu.SemaphoreType.DMA,
            pltpu.SemaphoreType.DMA,
            pltpu.SemaphoreType.DMA,
            pltpu.SemaphoreType.DMA,
        ],
    )
    def k(gidx_hbm, view_hbm, out_hbm,
          idx0, idx1, rows0, rows1, scores_v, semi0, semi1, semr0, semr1):
        wid = lax.axis_index("s") * _NC + lax.axis_index("c")
        base = wid * _PER_W
        lane = lax.iota(jnp.int32, 16)
        m15 = lane == 15
        idxb = (idx0, idx1)
        rowsb = (rows0, rows1)
        semi = (semi0, semi1)
        semr = (semr0, semr1)

        def idx_issue(b, ci):
            off = (base + ci * _E) * _G
            pltpu.async_copy(gidx_hbm.at[pl.ds(off, _ROWS)], idxb[b], semi[b])

        def idx_wait(b):
            pltpu.make_async_copy(
                gidx_hbm.at[pl.ds(0, _ROWS)], idxb[b], semi[b]).wait()

        def rows_issue(b):
            for j in range(_NSUB):
                pltpu.async_copy(
                    view_hbm.at[idxb[b].at[pl.ds(j * _SUB, _SUB)]],
                    rowsb[b].at[pl.ds(j * _SUB, _SUB)], semr[b])

        def rows_wait(b):
            pltpu.make_async_copy(
                view_hbm.at[pl.ds(0, _ROWS)], rowsb[b], semr[b]).wait()

        def compute(b, ci):
            rows = rowsb[b]

            def e_body(e, c2):
                ge = ci * _E + e
                r0 = e * _G
                base_idx = jnp.full((16,), ge * _J, jnp.int32)
                vt = [rows[r0, pl.ds(s * 16, 16)] for s in range(4)]
                for j in range(_J):
                    r = r0 + 1 + j
                    acc = vt[0] * rows[r, pl.ds(0, 16)]
                    for s in range(1, 4):
                        acc = acc + vt[s] * rows[r, pl.ds(s * 16, 16)]
                    if j == 0:
                        acc = -acc
                    sval = jnp.sum(acc)
                    csum = jnp.full((16,), sval, jnp.float32)
                    plsc.store_scatter(scores_v, [base_idx + j], csum,
                                       mask=m15)
                return c2

            lax.fori_loop(0, _E, e_body, 0)

        idx_issue(0, 0)
        idx_wait(0)
        rows_issue(0)
        idx_issue(1, 1)

        def pair_body(p, carry):
            ci = 2 * p
            idx_wait(1)
            rows_issue(1)
            rows_wait(0)
            idx_issue(0, ci + 2)
            compute(0, ci)
            idx_wait(0)
            rows_issue(0)
            rows_wait(1)
            idx_issue(1, ci + 3)
            compute(1, ci + 1)
            return carry

        lax.fori_loop(0, _NCHUNK // 2 - 1, pair_body, 0)
        idx_wait(1)
        rows_issue(1)
        rows_wait(0)
        compute(0, _NCHUNK - 2)
        rows_wait(1)
        compute(1, _NCHUNK - 1)

        pltpu.sync_copy(scores_v, out_hbm.at[pl.ds(base * _J, _PER_W * _J)])

    return k(gidx, view)


def _loss_tc(scores):
    flat = scores.reshape(_B * _J // 128, 128)

    def body(s_ref, o_ref):
        x = s_ref[...]
        term = -jnp.log(jax.nn.sigmoid(-x) + 1e-10)
        o_ref[0, 0] = jnp.sum(term) * (1.0 / _B)

    out = pl.pallas_call(
        body,
        out_shape=jax.ShapeDtypeStruct((1, 1), jnp.float32),
        out_specs=pl.BlockSpec(memory_space=pltpu.SMEM),
    )(flat)
    return out[0, 0]


def kernel(t, c, n, t_weight, c_weight):
    t = t.astype(jnp.int32)
    c = c.astype(jnp.int32)
    n = n.astype(jnp.int32)
    gidx = jnp.concatenate(
        [2 * t[:, None], 2 * c[:, None] + 1, 2 * n + 1], axis=1
    ).reshape(_B * _G)
    packed = _pack_tables(t_weight, c_weight)
    view = packed.reshape(2 * _V, _D)
    scores = _sc_scores(gidx, view)
    return _loss_tc(scores)


# speedup vs baseline: 14.1528x; 1.2184x over previous
import functools

import jax
import jax.numpy as jnp
from jax import lax
from jax.experimental import pallas as pl
from jax.experimental.pallas import tpu as pltpu
from jax.experimental.pallas import tpu_sc as plsc

_V = 1000000
_D = 64
_B = 16384
_K = 20
_J = _K + 1
_G = _J + 1

_NC = 2
_NS = 16
_NW = _NC * _NS
_PER_W = _B // _NW
_E = 32
_NCHUNK = _PER_W // _E
_ROWS = _E * _G
_SUB = 88
_NSUB = _ROWS // _SUB

_BLKI = 8192


def _pack_tables(t_wt, c_wt):
    tT = t_wt.T
    cT = c_wt.T

    def body(t_in, c_in, o):
        x = jnp.concatenate([t_in[...], c_in[...]], axis=0)
        o[...] = x.T

    in_spec = pl.BlockSpec((_D, _BLKI), lambda i: (0, i))
    out_spec = pl.BlockSpec((_BLKI, 2 * _D), lambda i: (i, 0))
    return pl.pallas_call(
        body,
        grid=((_V + _BLKI - 1) // _BLKI,),
        in_specs=[in_spec, in_spec],
        out_specs=out_spec,
        out_shape=jax.ShapeDtypeStruct((_V, 2 * _D), jnp.float32),
    )(tT, cT)


def _sc_scores(gidx, view):
    mesh = plsc.VectorSubcoreMesh(core_axis_name="c", subcore_axis_name="s")

    @functools.partial(
        pl.kernel,
        mesh=mesh,
        compiler_params=pltpu.CompilerParams(
            needs_layout_passes=False, use_tc_tiling_on_sc=False),
        out_type=jax.ShapeDtypeStruct((_B * _J,), jnp.float32),
        scratch_types=[
            pltpu.VMEM((_ROWS,), jnp.int32),
            pltpu.VMEM((_ROWS,), jnp.int32),
            pltpu.VMEM((_ROWS, _D), jnp.float32),
            pltpu.VMEM((_ROWS, _D), jnp.float32),
            pltpu.VMEM((_PER_W * _J,), jnp.float32),
            pltpu.SemaphoreType.DMA,
            pltpu.SemaphoreType.DMA,
            pltpu.SemaphoreType.DMA,
            pltpu.SemaphoreType.DMA,
        ],
    )
    def k(gidx_hbm, view_hbm, out_hbm,
          idx0, idx1, rows0, rows1, scores_v, semi0, semi1, semr0, semr1):
        wid = lax.axis_index("s") * _NC + lax.axis_index("c")
        base = wid * _PER_W
        lane = lax.iota(jnp.int32, 16)
        m15 = lane == 15
        idxb = (idx0, idx1)
        rowsb = (rows0, rows1)
        semi = (semi0, semi1)
        semr = (semr0, semr1)

        def idx_issue(b, ci):
            off = (base + ci * _E) * _G
            pltpu.async_copy(gidx_hbm.at[pl.ds(off, _ROWS)], idxb[b], semi[b])

        def idx_wait(b):
            pltpu.make_async_copy(
                gidx_hbm.at[pl.ds(0, _ROWS)], idxb[b], semi[b]).wait()

        def rows_issue(b):
            for j in range(_NSUB):
                pltpu.async_copy(
                    view_hbm.at[idxb[b].at[pl.ds(j * _SUB, _SUB)]],
                    rowsb[b].at[pl.ds(j * _SUB, _SUB)], semr[b])

        def rows_wait(b):
            pltpu.make_async_copy(
                view_hbm.at[pl.ds(0, _ROWS)], rowsb[b], semr[b]).wait()

        def compute(b, ci):
            rows = rowsb[b]

            @plsc.parallel_loop(0, _E, 1, unroll=2)
            def e_body(e):
                ge = ci * _E + e
                r0 = e * _G
                base_idx = jnp.full((16,), ge * _J, jnp.int32)
                vt = [rows[r0, pl.ds(s * 16, 16)] for s in range(4)]
                for j in range(_J):
                    r = r0 + 1 + j
                    acc = vt[0] * rows[r, pl.ds(0, 16)]
                    for s in range(1, 4):
                        acc = acc + vt[s] * rows[r, pl.ds(s * 16, 16)]
                    if j == 0:
                        acc = -acc
                    sval = jnp.sum(acc)
                    csum = jnp.full((16,), sval, jnp.float32)
                    plsc.store_scatter(scores_v, [base_idx + j], csum,
                                       mask=m15)

        idx_issue(0, 0)
        idx_wait(0)
        rows_issue(0)
        idx_issue(1, 1)

        def pair_body(p, carry):
            ci = 2 * p
            idx_wait(1)
            rows_issue(1)
            rows_wait(0)
            idx_issue(0, ci + 2)
            compute(0, ci)
            idx_wait(0)
            rows_issue(0)
            rows_wait(1)
            idx_issue(1, ci + 3)
            compute(1, ci + 1)
            return carry

        lax.fori_loop(0, _NCHUNK // 2 - 1, pair_body, 0)
        idx_wait(1)
        rows_issue(1)
        rows_wait(0)
        compute(0, _NCHUNK - 2)
        rows_wait(1)
        compute(1, _NCHUNK - 1)

        pltpu.sync_copy(scores_v, out_hbm.at[pl.ds(base * _J, _PER_W * _J)])

    return k(gidx, view)


def _loss_tc(scores):
    flat = scores.reshape(_B * _J // 128, 128)

    def body(s_ref, o_ref):
        x = s_ref[...]
        term = -jnp.log(jax.nn.sigmoid(-x) + 1e-10)
        o_ref[0, 0] = jnp.sum(term) * (1.0 / _B)

    out = pl.pallas_call(
        body,
        out_shape=jax.ShapeDtypeStruct((1, 1), jnp.float32),
        out_specs=pl.BlockSpec(memory_space=pltpu.SMEM),
    )(flat)
    return out[0, 0]


def kernel(t, c, n, t_weight, c_weight):
    t = t.astype(jnp.int32)
    c = c.astype(jnp.int32)
    n = n.astype(jnp.int32)
    gidx = jnp.concatenate(
        [2 * t[:, None], 2 * c[:, None] + 1, 2 * n + 1], axis=1
    ).reshape(_B * _G)
    packed = _pack_tables(t_weight, c_weight)
    view = packed.reshape(2 * _V, _D)
    scores = _sc_scores(gidx, view)
    return _loss_tc(scores)
